# Initial kernel scaffold; baseline (speedup 1.0000x reference)
import jax
import jax.numpy as jnp
from jax.experimental import pallas as pl


def kernel(x, preds, labels, centers):
    raise NotImplementedError("write your pallas kernel here")


# speedup vs baseline: 1.8910x; 1.8910x over previous
import functools

import jax
import jax.numpy as jnp
from jax import lax
from jax.experimental import pallas as pl
from jax.experimental.pallas import tpu as pltpu
from jax.experimental.pallas import tpu_sc as plsc

B = 4096
C = 1000
D = 512
EPS = 1e-6

BLK = 256


def _argmax_body(preds_ref, labels_ref, out_ref):
    p = preds_ref[...]
    lbl = labels_ref[...]
    col = lax.broadcasted_iota(jnp.int32, (BLK, C), 1)
    masked = jnp.where(col == lbl, -jnp.inf, p)
    out_ref[...] = jnp.argmax(masked, axis=1).astype(jnp.int32)


def _masked_argmax(preds, labels2d):
    return pl.pallas_call(
        _argmax_body,
        grid=(B // BLK,),
        in_specs=[
            pl.BlockSpec((BLK, C), lambda i: (i, 0)),
            pl.BlockSpec((BLK, 1), lambda i: (i, 0)),
        ],
        out_specs=pl.BlockSpec((BLK,), lambda i: (i,)),
        out_shape=jax.ShapeDtypeStruct((B,), jnp.int32),
    )(preds, labels2d)


NC = 2
NS = 16
NW = NC * NS
RPW = B // NW
CH = 32


def _sc_distances(x, labels, adv, centers):
    mesh = plsc.VectorSubcoreMesh(
        core_axis_name="c", subcore_axis_name="s",
        num_cores=NC, num_subcores=NS)

    @functools.partial(
        pl.kernel,
        mesh=mesh,
        compiler_params=pltpu.CompilerParams(needs_layout_passes=False),
        out_type=(
            jax.ShapeDtypeStruct((B,), jnp.float32),
            jax.ShapeDtypeStruct((B,), jnp.float32),
        ),
        scratch_types=[
            pltpu.VMEM((CH,), jnp.int32),
            pltpu.VMEM((CH,), jnp.int32),
            pltpu.VMEM((CH, D), jnp.float32),
            pltpu.VMEM((CH, D), jnp.float32),
            pltpu.VMEM((CH, D), jnp.float32),
            pltpu.VMEM((RPW,), jnp.float32),
            pltpu.VMEM((RPW,), jnp.float32),
            pltpu.SemaphoreType.DMA,
            pltpu.SemaphoreType.DMA,
        ],
    )
    def sc_kernel(x_hbm, lab_hbm, adv_hbm, cen_hbm, dap_hbm, dan_hbm,
                  lab_v, adv_v, x_v, pos_v, neg_v, dap_v, dan_v, semp, semn):
        wid = lax.axis_index("s") * NC + lax.axis_index("c")
        base = wid * RPW

        def chunk_body(ci, carry):
            row0 = base + ci * CH
            pltpu.sync_copy(lab_hbm.at[pl.ds(row0, CH)], lab_v)
            pltpu.sync_copy(adv_hbm.at[pl.ds(row0, CH)], adv_v)
            cp = pltpu.async_copy(cen_hbm.at[lab_v], pos_v, semp)
            cn = pltpu.async_copy(cen_hbm.at[adv_v], neg_v, semn)
            pltpu.sync_copy(x_hbm.at[pl.ds(row0, CH), :], x_v)
            cp.wait()
            cn.wait()

            lane0 = lax.iota(jnp.int32, 16) == 0

            def row_body(r, carry2):
                accp = jnp.zeros((16,), jnp.float32)
                accn = jnp.zeros((16,), jnp.float32)
                for j in range(D // 16):
                    xv = x_v[r, pl.ds(j * 16, 16)]
                    pv = pos_v[r, pl.ds(j * 16, 16)]
                    nv = neg_v[r, pl.ds(j * 16, 16)]
                    dp = xv - pv + EPS
                    dn = xv - nv + EPS
                    accp = accp + dp * dp
                    accn = accn + dn * dn
                dst = jnp.full((16,), ci * CH + r, jnp.int32)
                plsc.store_scatter(dap_v, [dst],
                                   jnp.full((16,), jnp.sum(accp), jnp.float32),
                                   mask=lane0)
                plsc.store_scatter(dan_v, [dst],
                                   jnp.full((16,), jnp.sum(accn), jnp.float32),
                                   mask=lane0)
                return carry2

            lax.fori_loop(0, CH, row_body, 0)
            return carry

        lax.fori_loop(0, RPW // CH, chunk_body, 0)
        pltpu.sync_copy(dap_v, dap_hbm.at[pl.ds(base, RPW)])
        pltpu.sync_copy(dan_v, dan_hbm.at[pl.ds(base, RPW)])

    return sc_kernel(x, labels, adv, centers)


def _finish_body(dap_ref, dan_ref, out_ref):
    d_ap = jnp.sqrt(dap_ref[...])
    d_an = jnp.sqrt(dan_ref[...])
    terms = jnp.maximum(d_ap - d_an + 1.0, 0.0)
    out_ref[...] = jnp.broadcast_to(jnp.sum(terms) * (1.0 / B), (1, 1))


def _finish(dap_sq, dan_sq):
    return pl.pallas_call(
        _finish_body,
        out_shape=jax.ShapeDtypeStruct((1, 1), jnp.float32),
    )(dap_sq.reshape(32, B // 32), dan_sq.reshape(32, B // 32))


def kernel(x, preds, labels, centers):
    labels = labels.astype(jnp.int32)
    adv = _masked_argmax(preds, labels.reshape(B, 1))
    dap_sq, dan_sq = _sc_distances(x, labels, adv, centers)
    loss = _finish(dap_sq, dan_sq)
    return loss.reshape(())
